# baseline (speedup 1.0000x reference)
import functools

import jax
import jax.numpy as jnp
from jax import lax
from jax.experimental import pallas as pl
from jax.experimental.pallas import tpu as pltpu
from jax.experimental.pallas import tpu_sc as plsc

N = 10000
E = 320000
NPAD = 10240
DISCARD = N + 16
K = 1024
NW = 32
W_CH = 10
NCHUNKS = NW * W_CH
RPT = NPAD // 16
RB = 4
RL = 2
DW = 4
BN = 1000
GRID = N // BN
V = 1.0


def _sc_mesh():
    return plsc.VectorSubcoreMesh(core_axis_name="c", subcore_axis_name="s",
                                  num_cores=2, num_subcores=16)


def _make_deg():
    @functools.partial(
        pl.kernel,
        out_type=jax.ShapeDtypeStruct((2, 2, NPAD, 16), jnp.float32),
        mesh=_sc_mesh(),
        compiler_params=pltpu.CompilerParams(use_tc_tiling_on_sc=False),
        scratch_types=[
            pltpu.VMEM((W_CH, K), jnp.int32),
            pltpu.VMEM((W_CH, K), jnp.int32),
            pltpu.VMEM((K, 16), jnp.float32),
            pltpu.VMEM((RPT, 16), jnp.float32),
            pltpu.VMEM_SHARED((NPAD, 16), jnp.float32),
            pltpu.SemaphoreType.DMA,
        ],
    )
    def deg(src_hbm, dst_hbm, ones_hbm, zero_hbm, out_hbm,
            sv, dv, ones_v, zbuf, acc, ssem):
        c = lax.axis_index("c")
        s = lax.axis_index("s")
        w = c * 16 + s
        pltpu.sync_copy(ones_hbm, ones_v)
        pltpu.sync_copy(zero_hbm, zbuf)
        pltpu.sync_copy(src_hbm.at[pl.ds(w * W_CH, W_CH)], sv)
        pltpu.sync_copy(dst_hbm.at[pl.ds(w * W_CH, W_CH)], dv)
        for slot, idx in ((0, sv), (1, dv)):
            pltpu.sync_copy(zbuf, acc.at[pl.ds(s * RPT, RPT)])
            plsc.subcore_barrier()

            def body(j, carry, idx=idx):
                pltpu.async_copy(ones_v, acc.at[idx.at[j]], ssem, add=True)

                @pl.when(j >= DW)
                def _():
                    pltpu.make_async_copy(ones_v, acc.at[idx.at[j - DW]],
                                          ssem).wait()

                return carry

            lax.fori_loop(0, W_CH, body, 0)

            def drain(j, carry, idx=idx):
                pltpu.make_async_copy(ones_v, acc.at[idx.at[j]], ssem).wait()
                return carry

            lax.fori_loop(W_CH - DW, W_CH, drain, 0)
            plsc.subcore_barrier()
            pltpu.sync_copy(acc.at[pl.ds(s * RPT, RPT)],
                            out_hbm.at[c, slot, pl.ds(s * RPT, RPT)])

    return deg


def _make_agg(D, NQ, stage):
    scratch = [
        pltpu.VMEM((W_CH, K), jnp.int32),
        pltpu.VMEM((W_CH, K), jnp.int32),
        pltpu.VMEM((RB, K, D), jnp.float32),
        pltpu.VMEM((RPT, D), jnp.float32),
        pltpu.VMEM_SHARED((NPAD, D), jnp.float32),
        pltpu.SemaphoreType.DMA,
        pltpu.SemaphoreType.DMA,
    ]
    if stage:
        scratch.insert(5, pltpu.VMEM_SHARED((NPAD, D), jnp.float32))

    @functools.partial(
        pl.kernel,
        out_type=jax.ShapeDtypeStruct((NQ, 2, NPAD, D), jnp.float32),
        mesh=_sc_mesh(),
        compiler_params=pltpu.CompilerParams(use_tc_tiling_on_sc=False),
        scratch_types=scratch,
    )
    def agg(*refs):
        h_hbms = refs[:NQ]
        if stage:
            (src_hbm, dst_hbm, zero_hbm, out_hbm,
             sv, dv, rows, zbuf, acc, hst, gsem, ssem) = refs[NQ:]
        else:
            (src_hbm, dst_hbm, zero_hbm, out_hbm,
             sv, dv, rows, zbuf, acc, gsem, ssem) = refs[NQ:]
        c = lax.axis_index("c")
        s = lax.axis_index("s")
        w = c * 16 + s
        pltpu.sync_copy(zero_hbm, zbuf)
        pltpu.sync_copy(src_hbm.at[pl.ds(w * W_CH, W_CH)], sv)
        pltpu.sync_copy(dst_hbm.at[pl.ds(w * W_CH, W_CH)], dv)
        for qi in range(NQ):
            h_hbm = h_hbms[qi]
            pltpu.sync_copy(zbuf, acc.at[pl.ds(s * RPT, RPT)])
            if stage:
                pltpu.sync_copy(h_hbm.at[pl.ds(s * (N // 16), N // 16)],
                                hst.at[pl.ds(s * (N // 16), N // 16)])
                h_src = hst
            else:
                h_src = h_hbm
            plsc.subcore_barrier()
            for t in range(RL):
                pltpu.async_copy(h_src.at[sv.at[t]], rows.at[t], gsem)

            def body(j, carry, h_src=h_src):
                slot = lax.rem(j, RB)
                pltpu.make_async_copy(h_src.at[sv.at[j]], rows.at[slot],
                                      gsem).wait()
                pltpu.async_copy(rows.at[slot], acc.at[dv.at[j]], ssem,
                                 add=True)

                @pl.when(j >= RB - RL)
                def _():
                    pltpu.make_async_copy(rows.at[lax.rem(j - (RB - RL), RB)],
                                          acc.at[dv.at[j - (RB - RL)]],
                                          ssem).wait()

                @pl.when(j + RL < W_CH)
                def _(h_src=h_src):
                    pltpu.async_copy(h_src.at[sv.at[j + RL]],
                                     rows.at[lax.rem(j + RL, RB)], gsem)

                return carry

            lax.fori_loop(0, W_CH, body, 0)

            def sdrain(j, carry):
                pltpu.make_async_copy(rows.at[lax.rem(j, RB)],
                                      acc.at[dv.at[j]], ssem).wait()
                return carry

            lax.fori_loop(W_CH - (RB - RL), W_CH, sdrain, 0)
            plsc.subcore_barrier()
            pltpu.sync_copy(acc.at[pl.ds(s * RPT, RPT)],
                            out_hbm.at[qi, c, pl.ds(s * RPT, RPT)])

    return agg


_get_deg = functools.cache(_make_deg)
_get_agg = functools.cache(_make_agg)


def _dot(a, b):
    return jnp.dot(a, b, preferred_element_type=jnp.float32)


def _leaky(v):
    return jnp.where(v >= 0, v, 0.01 * v)


def _softmax_rows(v):
    m = jnp.max(v, axis=1, keepdims=True)
    e = jnp.exp(v - m)
    return e / jnp.sum(e, axis=1, keepdims=True)


def _l2n_rows(v):
    n = jnp.sqrt(jnp.sum(v * v, axis=1, keepdims=True))
    return v / jnp.maximum(n, 1e-12)


def _row_spec(d):
    return pl.BlockSpec((BN, d), lambda i: (i, 0))


def _full_spec(shape):
    nd = len(shape)
    return pl.BlockSpec(shape, lambda i: (0,) * nd)


def _parts_spec(d):
    return pl.BlockSpec((2, BN, d), lambda i: (0, i, 0))


def _parts8_spec():
    return pl.BlockSpec((8, 2, BN, 16), lambda i: (0, 0, i, 0))


def _ae_body(x_ref, e1w, e1b, e2w, e2b, e3w, e3b, zw, zb,
             d1w, d1b, d2w, d2b, d3w, d3b, xw, xb,
             t1_o, t2_o, t3_o, z_o, xbar_o):
    x = x_ref[...]
    t1 = jnp.maximum(_dot(x, e1w[...]) + e1b[...], 0.0)
    t2 = jnp.maximum(_dot(t1, e2w[...]) + e2b[...], 0.0)
    t3 = jnp.maximum(_dot(t2, e3w[...]) + e3b[...], 0.0)
    z = _dot(t3, zw[...]) + zb[...]
    d1 = jnp.maximum(_dot(z, d1w[...]) + d1b[...], 0.0)
    d2 = jnp.maximum(_dot(d1, d2w[...]) + d2b[...], 0.0)
    d3 = jnp.maximum(_dot(d2, d3w[...]) + d3b[...], 0.0)
    xbar = _dot(d3, xw[...]) + xb[...]
    t1_o[...] = t1
    t2_o[...] = t2
    t3_o[...] = t3
    z_o[...] = z
    xbar_o[...] = xbar


def _ae_call(x, p):
    ws = [p['enc1_W'], p['enc1_b'].reshape(1, -1),
          p['enc2_W'], p['enc2_b'].reshape(1, -1),
          p['enc3_W'], p['enc3_b'].reshape(1, -1),
          p['z_W'], p['z_b'].reshape(1, -1),
          p['dec1_W'], p['dec1_b'].reshape(1, -1),
          p['dec2_W'], p['dec2_b'].reshape(1, -1),
          p['dec3_W'], p['dec3_b'].reshape(1, -1),
          p['xbar_W'], p['xbar_b'].reshape(1, -1)]
    return pl.pallas_call(
        _ae_body,
        grid=(GRID,),
        in_specs=[_row_spec(128)] + [_full_spec(w.shape) for w in ws],
        out_specs=(_row_spec(128), _row_spec(128), _row_spec(128),
                   _row_spec(16), _row_spec(128)),
        out_shape=(jax.ShapeDtypeStruct((N, 128), jnp.float32),
                   jax.ShapeDtypeStruct((N, 128), jnp.float32),
                   jax.ShapeDtypeStruct((N, 128), jnp.float32),
                   jax.ShapeDtypeStruct((N, 16), jnp.float32),
                   jax.ShapeDtypeStruct((N, 128), jnp.float32)),
    )(x, *ws)


def _prep_body(dpo_ref, dpi_ref, x_ref, w_ref, outn_o, inn_o, *h_os):
    a = dpo_ref[...]
    b = dpi_ref[...]
    od = a[0, :, 0:1] + a[1, :, 0:1]
    idg = b[0, :, 0:1] + b[1, :, 0:1]
    on = jnp.where(od > 0, lax.rsqrt(od), 0.0)
    inn = jnp.where(idg > 0, lax.rsqrt(idg), 0.0)
    outn_o[...] = on
    inn_o[...] = inn
    h = _dot(x_ref[...], w_ref[...]) * on
    for q, ref in enumerate(h_os):
        ref[...] = h[:, q * 16:(q + 1) * 16]


def _prep_call(dpo, dpi, x, w):
    q16 = jax.ShapeDtypeStruct((N, 16), jnp.float32)
    return pl.pallas_call(
        _prep_body,
        grid=(GRID,),
        in_specs=[_parts_spec(16), _parts_spec(16), _row_spec(128),
                  _full_spec(w.shape)],
        out_specs=(_row_spec(1), _row_spec(1)) + (_row_spec(16),) * 8,
        out_shape=(jax.ShapeDtypeStruct((N, 1), jnp.float32),
                   jax.ShapeDtypeStruct((N, 1), jnp.float32)) + (q16,) * 8,
    )(dpo, dpi, x, w)


def _make_layer_body(dout):
    def body(parts_ref, inn_ref, outn_ref, tra_ref, wt_ref, wz_ref,
             b_ref, gw_ref, z_o, *hs_os):
        a = parts_ref[...]
        agg = jnp.concatenate([a[q, 0] + a[q, 1] for q in range(8)], axis=1)
        agg = agg * inn_ref[...]
        zi = _leaky(agg)
        t = tra_ref[...]
        logits = _leaky(_dot(t, wt_ref[...]) + _dot(zi, wz_ref[...])
                        + b_ref[...])
        m = _l2n_rows(_softmax_rows(logits))
        g = m[:, 0:1] * zi + m[:, 1:2] * t
        hs = _dot(g, gw_ref[...]) * outn_ref[...]
        z_o[...] = zi
        if dout == 128:
            for q, ref in enumerate(hs_os):
                ref[...] = hs[:, q * 16:(q + 1) * 16]
        else:
            hs_os[0][...] = hs

    return body


def _layer_call(parts, inn, outn, tra, wt, wz, b, gw, dout):
    if dout == 128:
        hs_shape = (jax.ShapeDtypeStruct((N, 16), jnp.float32),) * 8
        hs_spec = (_row_spec(16),) * 8
    else:
        hs_shape = (jax.ShapeDtypeStruct((N, dout), jnp.float32),)
        hs_spec = (_row_spec(dout),)
    return pl.pallas_call(
        _make_layer_body(dout),
        grid=(GRID,),
        in_specs=[_parts8_spec(), _row_spec(1), _row_spec(1), _row_spec(128),
                  _full_spec(wt.shape), _full_spec(wz.shape),
                  _full_spec(b.shape), _full_spec(gw.shape)],
        out_specs=(_row_spec(128),) + hs_spec,
        out_shape=(jax.ShapeDtypeStruct((N, 128), jnp.float32),) + hs_shape,
    )(parts, inn, outn, tra, wt, wz, b, gw)


def _a5_body(parts_ref, inn_ref, outn_ref, z1_ref, z2_ref, z3_ref, z_ref,
             wl1, wl2, wl3, wl4, wl5, bl, g1, g2, g3, g4, g5, hs_o):
    a = parts_ref[...]
    z4 = _leaky((a[0] + a[1]) * inn_ref[...])
    z1 = z1_ref[...]
    z2 = z2_ref[...]
    z3 = z3_ref[...]
    zz = z_ref[...]
    logits = (_dot(z1, wl1[...]) + _dot(z2, wl2[...]) + _dot(z3, wl3[...])
              + _dot(z4, wl4[...]) + _dot(zz, wl5[...]) + bl[...])
    u = _l2n_rows(_softmax_rows(_leaky(logits)))
    h = (_dot(u[:, 0:1] * z1, g1[...]) + _dot(u[:, 1:2] * z2, g2[...])
         + _dot(u[:, 2:3] * z3, g3[...]) + _dot(u[:, 3:4] * z4, g4[...])
         + _dot(u[:, 4:5] * zz, g5[...]))
    hs_o[...] = h * outn_ref[...]


def _a5_call(parts, inn, outn, z1, z2, z3, z, wls, bl, gs):
    return pl.pallas_call(
        _a5_body,
        grid=(GRID,),
        in_specs=[_parts_spec(16), _row_spec(1), _row_spec(1),
                  _row_spec(128), _row_spec(128), _row_spec(128),
                  _row_spec(16)]
                 + [_full_spec(w.shape) for w in wls]
                 + [_full_spec(bl.shape)]
                 + [_full_spec(g.shape) for g in gs],
        out_specs=_row_spec(16),
        out_shape=jax.ShapeDtypeStruct((N, 16), jnp.float32),
    )(parts, inn, outn, z1, z2, z3, z, *wls, bl, *gs)


def _a6_body(parts_ref, inn_ref, z_ref, ct_ref, pred_o, q_o, qcol_o):
    i = pl.program_id(0)
    a = parts_ref[...]
    h = (a[0] + a[1]) * inn_ref[...]
    pred_o[...] = _softmax_rows(h)
    zb = z_ref[...]
    ct = ct_ref[...]
    zn = jnp.sum(zb * zb, axis=1, keepdims=True)
    cn = jnp.sum(ct * ct, axis=0, keepdims=True)
    dist = zn + cn - 2.0 * _dot(zb, ct)
    qu = 1.0 / (1.0 + dist / V)
    q = qu / jnp.sum(qu, axis=1, keepdims=True)
    q_o[...] = q

    @pl.when(i == 0)
    def _():
        qcol_o[...] = jnp.zeros_like(qcol_o)

    qcol_o[...] += jnp.sum(q, axis=0, keepdims=True)


def _a6_call(parts, inn, z, ct):
    return pl.pallas_call(
        _a6_body,
        grid=(GRID,),
        in_specs=[_parts_spec(16), _row_spec(1), _row_spec(16),
                  _full_spec(ct.shape)],
        out_specs=(_row_spec(16), _row_spec(16),
                   pl.BlockSpec((1, 16), lambda i: (0, 0))),
        out_shape=(jax.ShapeDtypeStruct((N, 16), jnp.float32),
                   jax.ShapeDtypeStruct((N, 16), jnp.float32),
                   jax.ShapeDtypeStruct((1, 16), jnp.float32)),
    )(parts, inn, z, ct)


def _p_body(q_ref, qcol_ref, p_o):
    q = q_ref[...]
    w = q * q / qcol_ref[...]
    p_o[...] = w / jnp.sum(w, axis=1, keepdims=True)


def _p_call(q, qcol):
    return pl.pallas_call(
        _p_body,
        grid=(GRID,),
        in_specs=[_row_spec(16), _full_spec((1, 16))],
        out_specs=_row_spec(16),
        out_shape=jax.ShapeDtypeStruct((N, 16), jnp.float32),
    )(q, qcol)


def kernel(x, edge_index, params):
    p = params
    npad_e = NCHUNKS * K - E
    pad_discard = jnp.full((npad_e,), DISCARD, jnp.int32)
    src_agg = jnp.concatenate([edge_index[0], jnp.zeros((npad_e,), jnp.int32)])
    src_deg = jnp.concatenate([edge_index[0], pad_discard]).reshape(NCHUNKS, K)
    dst2d = jnp.concatenate([edge_index[1], pad_discard]).reshape(NCHUNKS, K)
    src2d = src_agg.reshape(NCHUNKS, K)
    ones_sc = jnp.ones((K, 16), jnp.float32)
    z16 = jnp.zeros((RPT, 16), jnp.float32)

    deg_parts = _get_deg()(src_deg, dst2d, ones_sc, z16)
    dpo = deg_parts[:, 0]
    dpi = deg_parts[:, 1]

    tra1, tra2, tra3, z, x_bar = _ae_call(x, p)
    out_n, in_n, *h1q = _prep_call(dpo, dpi, x, p['gcn1_W'])

    parts1 = _get_agg(16, 8, True)(*h1q, src2d, dst2d, z16)
    z1, *h2q = _layer_call(parts1, in_n, out_n, tra1,
                           p['mlp1_W'][:128], p['mlp1_W'][128:],
                           p['mlp1_b'].reshape(1, 2), p['gcn2_W'], 128)
    parts2 = _get_agg(16, 8, True)(*h2q, src2d, dst2d, z16)
    z2, *h3q = _layer_call(parts2, in_n, out_n, tra2,
                           p['mlp2_W'][:128], p['mlp2_W'][128:],
                           p['mlp2_b'].reshape(1, 2), p['gcn3_W'], 128)
    parts3 = _get_agg(16, 8, True)(*h3q, src2d, dst2d, z16)
    z3, h4s = _layer_call(parts3, in_n, out_n, tra3,
                          p['mlp3_W'][:128], p['mlp3_W'][128:],
                          p['mlp3_b'].reshape(1, 2), p['gcn4_W'], 16)
    parts4 = _get_agg(16, 1, False)(h4s, src2d, dst2d, z16)[0]

    wl = p['mlpL_W']
    g5 = p['gcn5_W']
    wls = [wl[0:128], wl[128:256], wl[256:384], wl[384:400], wl[400:416]]
    gs = [g5[0:128], g5[128:256], g5[256:384], g5[384:400], g5[400:416]]
    h5s = _a5_call(parts4, in_n, out_n, z1, z2, z3, z,
                   wls, p['mlpL_b'].reshape(1, 5), gs)
    parts5 = _get_agg(16, 1, False)(h5s, src2d, dst2d, z16)[0]

    predict, q, qcol = _a6_call(parts5, in_n, z, p['cluster'].T)
    p_out = _p_call(q, qcol)
    return (x_bar, q, predict, p_out)


# speedup vs baseline: 1.0382x; 1.0382x over previous
import functools

import jax
import jax.numpy as jnp
from jax import lax
from jax.experimental import pallas as pl
from jax.experimental.pallas import tpu as pltpu
from jax.experimental.pallas import tpu_sc as plsc

N = 10000
E = 320000
NPAD = 10240
DISCARD = N + 16
K = 512
NW = 32
W_CH = 20
NCHUNKS = NW * W_CH
RPT = NPAD // 16
RB = 4
RL = 2
DW = 4
BN = 1000
GRID = N // BN
V = 1.0


def _sc_mesh():
    return plsc.VectorSubcoreMesh(core_axis_name="c", subcore_axis_name="s",
                                  num_cores=2, num_subcores=16)


def _make_deg():
    @functools.partial(
        pl.kernel,
        out_type=jax.ShapeDtypeStruct((2, 2, NPAD, 16), jnp.float32),
        mesh=_sc_mesh(),
        compiler_params=pltpu.CompilerParams(use_tc_tiling_on_sc=False),
        scratch_types=[
            pltpu.VMEM((W_CH, K), jnp.int32),
            pltpu.VMEM((W_CH, K), jnp.int32),
            pltpu.VMEM((K, 16), jnp.float32),
            pltpu.VMEM((RPT, 16), jnp.float32),
            pltpu.VMEM_SHARED((NPAD, 16), jnp.float32),
            pltpu.SemaphoreType.DMA,
        ],
    )
    def deg(src_hbm, dst_hbm, ones_hbm, zero_hbm, out_hbm,
            sv, dv, ones_v, zbuf, acc, ssem):
        c = lax.axis_index("c")
        s = lax.axis_index("s")
        w = c * 16 + s
        pltpu.sync_copy(ones_hbm, ones_v)
        pltpu.sync_copy(zero_hbm, zbuf)
        pltpu.sync_copy(src_hbm.at[pl.ds(w * W_CH, W_CH)], sv)
        pltpu.sync_copy(dst_hbm.at[pl.ds(w * W_CH, W_CH)], dv)
        for slot, idx in ((0, sv), (1, dv)):
            pltpu.sync_copy(zbuf, acc.at[pl.ds(s * RPT, RPT)])
            plsc.subcore_barrier()

            def body(j, carry, idx=idx):
                pltpu.async_copy(ones_v, acc.at[idx.at[j]], ssem, add=True)

                @pl.when(j >= DW)
                def _():
                    pltpu.make_async_copy(ones_v, acc.at[idx.at[j - DW]],
                                          ssem).wait()

                return carry

            lax.fori_loop(0, W_CH, body, 0)

            def drain(j, carry, idx=idx):
                pltpu.make_async_copy(ones_v, acc.at[idx.at[j]], ssem).wait()
                return carry

            lax.fori_loop(W_CH - DW, W_CH, drain, 0)
            plsc.subcore_barrier()
            pltpu.sync_copy(acc.at[pl.ds(s * RPT, RPT)],
                            out_hbm.at[c, slot, pl.ds(s * RPT, RPT)])

    return deg


def _make_agg(D, NQ, stage):
    scratch = [
        pltpu.VMEM((W_CH, K), jnp.int32),
        pltpu.VMEM((W_CH, K), jnp.int32),
        pltpu.VMEM((RB, K, D), jnp.float32),
        pltpu.VMEM((RPT, D), jnp.float32),
        pltpu.VMEM_SHARED((NPAD, D), jnp.float32),
        pltpu.SemaphoreType.DMA,
        pltpu.SemaphoreType.DMA,
    ]
    if stage:
        scratch.insert(5, pltpu.VMEM_SHARED((NPAD, D), jnp.float32))
        scratch.append(pltpu.VMEM((2, N // 16, D), jnp.float32))
        scratch.append(pltpu.SemaphoreType.DMA)
        scratch.append(pltpu.SemaphoreType.DMA)

    @functools.partial(
        pl.kernel,
        out_type=jax.ShapeDtypeStruct((NQ, 2, NPAD, D), jnp.float32),
        mesh=_sc_mesh(),
        compiler_params=pltpu.CompilerParams(use_tc_tiling_on_sc=False),
        scratch_types=scratch,
    )
    def agg(*refs):
        h_hbms = refs[:NQ]
        if stage:
            (src_hbm, dst_hbm, zero_hbm, out_hbm,
             sv, dv, rows, zbuf, acc, hst, gsem, ssem,
             vfill, fsem, rosem) = refs[NQ:]
        else:
            (src_hbm, dst_hbm, zero_hbm, out_hbm,
             sv, dv, rows, zbuf, acc, gsem, ssem) = refs[NQ:]
        c = lax.axis_index("c")
        s = lax.axis_index("s")
        w = c * 16 + s
        nf = N // 16
        fsl = pl.ds(s * nf, nf)
        rsl = pl.ds(s * RPT, RPT)
        pltpu.sync_copy(zero_hbm, zbuf)
        pltpu.sync_copy(src_hbm.at[pl.ds(w * W_CH, W_CH)], sv)
        pltpu.sync_copy(dst_hbm.at[pl.ds(w * W_CH, W_CH)], dv)
        if stage:
            pltpu.async_copy(h_hbms[0].at[fsl], vfill.at[0], fsem)
        for qi in range(NQ):
            h_hbm = h_hbms[qi]
            if stage:
                pltpu.make_async_copy(h_hbm.at[fsl], vfill.at[qi % 2],
                                      fsem).wait()
                pltpu.sync_copy(vfill.at[qi % 2], hst.at[fsl])
                if qi + 1 < NQ:
                    pltpu.async_copy(h_hbms[qi + 1].at[fsl],
                                     vfill.at[(qi + 1) % 2], fsem)
                if qi >= 1:
                    pltpu.make_async_copy(acc.at[rsl],
                                          out_hbm.at[qi - 1, c, rsl],
                                          rosem).wait()
                pltpu.sync_copy(zbuf, acc.at[rsl])
                h_src = hst
            else:
                pltpu.sync_copy(zbuf, acc.at[rsl])
                h_src = h_hbm
            plsc.subcore_barrier()
            for t in range(RL):
                pltpu.async_copy(h_src.at[sv.at[t]], rows.at[t], gsem)

            def body(j, carry, h_src=h_src):
                slot = lax.rem(j, RB)
                pltpu.make_async_copy(h_src.at[sv.at[j]], rows.at[slot],
                                      gsem).wait()
                pltpu.async_copy(rows.at[slot], acc.at[dv.at[j]], ssem,
                                 add=True)

                @pl.when(j >= RB - RL)
                def _():
                    pltpu.make_async_copy(rows.at[lax.rem(j - (RB - RL), RB)],
                                          acc.at[dv.at[j - (RB - RL)]],
                                          ssem).wait()

                @pl.when(j + RL < W_CH)
                def _(h_src=h_src):
                    pltpu.async_copy(h_src.at[sv.at[j + RL]],
                                     rows.at[lax.rem(j + RL, RB)], gsem)

                return carry

            lax.fori_loop(0, W_CH, body, 0)

            def sdrain(j, carry):
                pltpu.make_async_copy(rows.at[lax.rem(j, RB)],
                                      acc.at[dv.at[j]], ssem).wait()
                return carry

            lax.fori_loop(W_CH - (RB - RL), W_CH, sdrain, 0)
            plsc.subcore_barrier()
            if stage:
                pltpu.async_copy(acc.at[rsl], out_hbm.at[qi, c, rsl], rosem)
            else:
                pltpu.sync_copy(acc.at[rsl], out_hbm.at[qi, c, rsl])
        if stage:
            pltpu.make_async_copy(acc.at[rsl], out_hbm.at[NQ - 1, c, rsl],
                                  rosem).wait()

    return agg


_get_deg = functools.cache(_make_deg)
_get_agg = functools.cache(_make_agg)


def _dot(a, b):
    return jnp.dot(a, b, preferred_element_type=jnp.float32)


def _leaky(v):
    return jnp.where(v >= 0, v, 0.01 * v)


def _softmax_rows(v):
    m = jnp.max(v, axis=1, keepdims=True)
    e = jnp.exp(v - m)
    return e / jnp.sum(e, axis=1, keepdims=True)


def _l2n_rows(v):
    n = jnp.sqrt(jnp.sum(v * v, axis=1, keepdims=True))
    return v / jnp.maximum(n, 1e-12)


def _row_spec(d):
    return pl.BlockSpec((BN, d), lambda i: (i, 0))


def _full_spec(shape):
    nd = len(shape)
    return pl.BlockSpec(shape, lambda i: (0,) * nd)


def _parts_spec(d):
    return pl.BlockSpec((2, BN, d), lambda i: (0, i, 0))


def _parts8_spec():
    return pl.BlockSpec((8, 2, BN, 16), lambda i: (0, 0, i, 0))


def _ae_body(x_ref, e1w, e1b, e2w, e2b, e3w, e3b, zw, zb,
             d1w, d1b, d2w, d2b, d3w, d3b, xw, xb,
             t1_o, t2_o, t3_o, z_o, xbar_o):
    x = x_ref[...]
    t1 = jnp.maximum(_dot(x, e1w[...]) + e1b[...], 0.0)
    t2 = jnp.maximum(_dot(t1, e2w[...]) + e2b[...], 0.0)
    t3 = jnp.maximum(_dot(t2, e3w[...]) + e3b[...], 0.0)
    z = _dot(t3, zw[...]) + zb[...]
    d1 = jnp.maximum(_dot(z, d1w[...]) + d1b[...], 0.0)
    d2 = jnp.maximum(_dot(d1, d2w[...]) + d2b[...], 0.0)
    d3 = jnp.maximum(_dot(d2, d3w[...]) + d3b[...], 0.0)
    xbar = _dot(d3, xw[...]) + xb[...]
    t1_o[...] = t1
    t2_o[...] = t2
    t3_o[...] = t3
    z_o[...] = z
    xbar_o[...] = xbar


def _ae_call(x, p):
    ws = [p['enc1_W'], p['enc1_b'].reshape(1, -1),
          p['enc2_W'], p['enc2_b'].reshape(1, -1),
          p['enc3_W'], p['enc3_b'].reshape(1, -1),
          p['z_W'], p['z_b'].reshape(1, -1),
          p['dec1_W'], p['dec1_b'].reshape(1, -1),
          p['dec2_W'], p['dec2_b'].reshape(1, -1),
          p['dec3_W'], p['dec3_b'].reshape(1, -1),
          p['xbar_W'], p['xbar_b'].reshape(1, -1)]
    return pl.pallas_call(
        _ae_body,
        grid=(GRID,),
        in_specs=[_row_spec(128)] + [_full_spec(w.shape) for w in ws],
        out_specs=(_row_spec(128), _row_spec(128), _row_spec(128),
                   _row_spec(16), _row_spec(128)),
        out_shape=(jax.ShapeDtypeStruct((N, 128), jnp.float32),
                   jax.ShapeDtypeStruct((N, 128), jnp.float32),
                   jax.ShapeDtypeStruct((N, 128), jnp.float32),
                   jax.ShapeDtypeStruct((N, 16), jnp.float32),
                   jax.ShapeDtypeStruct((N, 128), jnp.float32)),
    )(x, *ws)


def _prep_body(dpo_ref, dpi_ref, x_ref, w_ref, outn_o, inn_o, *h_os):
    a = dpo_ref[...]
    b = dpi_ref[...]
    od = a[0, :, 0:1] + a[1, :, 0:1]
    idg = b[0, :, 0:1] + b[1, :, 0:1]
    on = jnp.where(od > 0, lax.rsqrt(od), 0.0)
    inn = jnp.where(idg > 0, lax.rsqrt(idg), 0.0)
    outn_o[...] = on
    inn_o[...] = inn
    h = _dot(x_ref[...], w_ref[...]) * on
    for q, ref in enumerate(h_os):
        ref[...] = h[:, q * 16:(q + 1) * 16]


def _prep_call(dpo, dpi, x, w):
    q16 = jax.ShapeDtypeStruct((N, 16), jnp.float32)
    return pl.pallas_call(
        _prep_body,
        grid=(GRID,),
        in_specs=[_parts_spec(16), _parts_spec(16), _row_spec(128),
                  _full_spec(w.shape)],
        out_specs=(_row_spec(1), _row_spec(1)) + (_row_spec(16),) * 8,
        out_shape=(jax.ShapeDtypeStruct((N, 1), jnp.float32),
                   jax.ShapeDtypeStruct((N, 1), jnp.float32)) + (q16,) * 8,
    )(dpo, dpi, x, w)


def _make_layer_body(dout):
    def body(parts_ref, inn_ref, outn_ref, tra_ref, wt_ref, wz_ref,
             b_ref, gw_ref, z_o, *hs_os):
        a = parts_ref[...]
        agg = jnp.concatenate([a[q, 0] + a[q, 1] for q in range(8)], axis=1)
        agg = agg * inn_ref[...]
        zi = _leaky(agg)
        t = tra_ref[...]
        logits = _leaky(_dot(t, wt_ref[...]) + _dot(zi, wz_ref[...])
                        + b_ref[...])
        m = _l2n_rows(_softmax_rows(logits))
        g = m[:, 0:1] * zi + m[:, 1:2] * t
        hs = _dot(g, gw_ref[...]) * outn_ref[...]
        z_o[...] = zi
        if dout == 128:
            for q, ref in enumerate(hs_os):
                ref[...] = hs[:, q * 16:(q + 1) * 16]
        else:
            hs_os[0][...] = hs

    return body


def _layer_call(parts, inn, outn, tra, wt, wz, b, gw, dout):
    if dout == 128:
        hs_shape = (jax.ShapeDtypeStruct((N, 16), jnp.float32),) * 8
        hs_spec = (_row_spec(16),) * 8
    else:
        hs_shape = (jax.ShapeDtypeStruct((N, dout), jnp.float32),)
        hs_spec = (_row_spec(dout),)
    return pl.pallas_call(
        _make_layer_body(dout),
        grid=(GRID,),
        in_specs=[_parts8_spec(), _row_spec(1), _row_spec(1), _row_spec(128),
                  _full_spec(wt.shape), _full_spec(wz.shape),
                  _full_spec(b.shape), _full_spec(gw.shape)],
        out_specs=(_row_spec(128),) + hs_spec,
        out_shape=(jax.ShapeDtypeStruct((N, 128), jnp.float32),) + hs_shape,
    )(parts, inn, outn, tra, wt, wz, b, gw)


def _a5_body(parts_ref, inn_ref, outn_ref, z1_ref, z2_ref, z3_ref, z_ref,
             wl1, wl2, wl3, wl4, wl5, bl, g1, g2, g3, g4, g5, hs_o):
    a = parts_ref[...]
    z4 = _leaky((a[0] + a[1]) * inn_ref[...])
    z1 = z1_ref[...]
    z2 = z2_ref[...]
    z3 = z3_ref[...]
    zz = z_ref[...]
    logits = (_dot(z1, wl1[...]) + _dot(z2, wl2[...]) + _dot(z3, wl3[...])
              + _dot(z4, wl4[...]) + _dot(zz, wl5[...]) + bl[...])
    u = _l2n_rows(_softmax_rows(_leaky(logits)))
    h = (_dot(u[:, 0:1] * z1, g1[...]) + _dot(u[:, 1:2] * z2, g2[...])
         + _dot(u[:, 2:3] * z3, g3[...]) + _dot(u[:, 3:4] * z4, g4[...])
         + _dot(u[:, 4:5] * zz, g5[...]))
    hs_o[...] = h * outn_ref[...]


def _a5_call(parts, inn, outn, z1, z2, z3, z, wls, bl, gs):
    return pl.pallas_call(
        _a5_body,
        grid=(GRID,),
        in_specs=[_parts_spec(16), _row_spec(1), _row_spec(1),
                  _row_spec(128), _row_spec(128), _row_spec(128),
                  _row_spec(16)]
                 + [_full_spec(w.shape) for w in wls]
                 + [_full_spec(bl.shape)]
                 + [_full_spec(g.shape) for g in gs],
        out_specs=_row_spec(16),
        out_shape=jax.ShapeDtypeStruct((N, 16), jnp.float32),
    )(parts, inn, outn, z1, z2, z3, z, *wls, bl, *gs)


def _a6_body(parts_ref, inn_ref, z_ref, ct_ref, pred_o, q_o, qcol_o):
    i = pl.program_id(0)
    a = parts_ref[...]
    h = (a[0] + a[1]) * inn_ref[...]
    pred_o[...] = _softmax_rows(h)
    zb = z_ref[...]
    ct = ct_ref[...]
    zn = jnp.sum(zb * zb, axis=1, keepdims=True)
    cn = jnp.sum(ct * ct, axis=0, keepdims=True)
    dist = zn + cn - 2.0 * _dot(zb, ct)
    qu = 1.0 / (1.0 + dist / V)
    q = qu / jnp.sum(qu, axis=1, keepdims=True)
    q_o[...] = q

    @pl.when(i == 0)
    def _():
        qcol_o[...] = jnp.zeros_like(qcol_o)

    qcol_o[...] += jnp.sum(q, axis=0, keepdims=True)


def _a6_call(parts, inn, z, ct):
    return pl.pallas_call(
        _a6_body,
        grid=(GRID,),
        in_specs=[_parts_spec(16), _row_spec(1), _row_spec(16),
                  _full_spec(ct.shape)],
        out_specs=(_row_spec(16), _row_spec(16),
                   pl.BlockSpec((1, 16), lambda i: (0, 0))),
        out_shape=(jax.ShapeDtypeStruct((N, 16), jnp.float32),
                   jax.ShapeDtypeStruct((N, 16), jnp.float32),
                   jax.ShapeDtypeStruct((1, 16), jnp.float32)),
    )(parts, inn, z, ct)


def _p_body(q_ref, qcol_ref, p_o):
    q = q_ref[...]
    w = q * q / qcol_ref[...]
    p_o[...] = w / jnp.sum(w, axis=1, keepdims=True)


def _p_call(q, qcol):
    return pl.pallas_call(
        _p_body,
        grid=(GRID,),
        in_specs=[_row_spec(16), _full_spec((1, 16))],
        out_specs=_row_spec(16),
        out_shape=jax.ShapeDtypeStruct((N, 16), jnp.float32),
    )(q, qcol)


def kernel(x, edge_index, params):
    p = params
    npad_e = NCHUNKS * K - E
    pad_discard = jnp.full((npad_e,), DISCARD, jnp.int32)
    src_agg = jnp.concatenate([edge_index[0], jnp.zeros((npad_e,), jnp.int32)])
    src_deg = jnp.concatenate([edge_index[0], pad_discard]).reshape(NCHUNKS, K)
    dst2d = jnp.concatenate([edge_index[1], pad_discard]).reshape(NCHUNKS, K)
    src2d = src_agg.reshape(NCHUNKS, K)
    ones_sc = jnp.ones((K, 16), jnp.float32)
    z16 = jnp.zeros((RPT, 16), jnp.float32)

    deg_parts = _get_deg()(src_deg, dst2d, ones_sc, z16)
    dpo = deg_parts[:, 0]
    dpi = deg_parts[:, 1]

    tra1, tra2, tra3, z, x_bar = _ae_call(x, p)
    out_n, in_n, *h1q = _prep_call(dpo, dpi, x, p['gcn1_W'])

    parts1 = _get_agg(16, 8, True)(*h1q, src2d, dst2d, z16)
    z1, *h2q = _layer_call(parts1, in_n, out_n, tra1,
                           p['mlp1_W'][:128], p['mlp1_W'][128:],
                           p['mlp1_b'].reshape(1, 2), p['gcn2_W'], 128)
    parts2 = _get_agg(16, 8, True)(*h2q, src2d, dst2d, z16)
    z2, *h3q = _layer_call(parts2, in_n, out_n, tra2,
                           p['mlp2_W'][:128], p['mlp2_W'][128:],
                           p['mlp2_b'].reshape(1, 2), p['gcn3_W'], 128)
    parts3 = _get_agg(16, 8, True)(*h3q, src2d, dst2d, z16)
    z3, h4s = _layer_call(parts3, in_n, out_n, tra3,
                          p['mlp3_W'][:128], p['mlp3_W'][128:],
                          p['mlp3_b'].reshape(1, 2), p['gcn4_W'], 16)
    parts4 = _get_agg(16, 1, False)(h4s, src2d, dst2d, z16)[0]

    wl = p['mlpL_W']
    g5 = p['gcn5_W']
    wls = [wl[0:128], wl[128:256], wl[256:384], wl[384:400], wl[400:416]]
    gs = [g5[0:128], g5[128:256], g5[256:384], g5[384:400], g5[400:416]]
    h5s = _a5_call(parts4, in_n, out_n, z1, z2, z3, z,
                   wls, p['mlpL_b'].reshape(1, 5), gs)
    parts5 = _get_agg(16, 1, False)(h5s, src2d, dst2d, z16)[0]

    predict, q, qcol = _a6_call(parts5, in_n, z, p['cluster'].T)
    p_out = _p_call(q, qcol)
    return (x_bar, q, predict, p_out)
